# baseline (speedup 1.0000x reference)
import functools

import jax
import jax.numpy as jnp
from jax.experimental import pallas as pl
from jax.experimental.pallas import tpu as pltpu


def _transpose_cast_body(src_hbm, o_ref, buf, sems):
    m = pl.program_id(0)
    slot = jax.lax.rem(m, 2)

    @pl.when(m == 0)
    def _first():
        pltpu.make_async_copy(src_hbm.at[:, 0, :], buf.at[0], sems.at[0]).start()

    @pl.when(m < pl.num_programs(0) - 1)
    def _prefetch():
        nslot = jax.lax.rem(m + 1, 2)
        pltpu.make_async_copy(
            src_hbm.at[:, m + 1, :], buf.at[nslot], sems.at[nslot]
        ).start()

    pltpu.make_async_copy(src_hbm.at[:, m, :], buf.at[slot], sems.at[slot]).wait()
    o_ref[0] = buf[slot].astype(o_ref.dtype)


def _transpose_cast(rw3):
    D, M, C = rw3.shape
    return pl.pallas_call(
        _transpose_cast_body,
        grid=(M,),
        in_specs=[pl.BlockSpec(memory_space=pl.ANY)],
        out_specs=pl.BlockSpec((1, D, C), lambda m: (m, 0, 0)),
        out_shape=jax.ShapeDtypeStruct((M, D, C), jnp.bfloat16),
        scratch_shapes=[
            pltpu.VMEM((2, D, C), jnp.float32),
            pltpu.SemaphoreType.DMA((2,)),
        ],
        compiler_params=pltpu.CompilerParams(
            dimension_semantics=("arbitrary",),
            vmem_limit_bytes=56 * 1024 * 1024,
        ),
    )(rw3)


def _fused_body(x_ref, w_ref, b_ref, rw_ref, rb_ref, o_ref):
    m = pl.program_id(1)
    xb = x_ref[...]
    logits = jnp.dot(xb, w_ref[0], preferred_element_type=jnp.float32)
    fusew = jnp.dot(xb, rw_ref[0], preferred_element_type=jnp.float32)
    term = (logits + b_ref[0]) * (fusew + rb_ref[0])

    @pl.when(m == 0)
    def _init():
        o_ref[...] = term

    @pl.when(m != 0)
    def _acc():
        o_ref[...] += term


def _fused_call(xc, mw, mb, rw, rb, bB):
    B, D = xc.shape
    M, _, C = mw.shape
    grid = (B // bB, M)
    return pl.pallas_call(
        _fused_body,
        grid=grid,
        in_specs=[
            pl.BlockSpec((bB, D), lambda b, m: (b, 0)),
            pl.BlockSpec((1, D, C), lambda b, m: (m, 0, 0)),
            pl.BlockSpec((1, 1, C), lambda b, m: (m, 0, 0)),
            pl.BlockSpec((1, D, C), lambda b, m: (m, 0, 0)),
            pl.BlockSpec((1, 1, C), lambda b, m: (m, 0, 0)),
        ],
        out_specs=pl.BlockSpec((bB, C), lambda b, m: (b, 0)),
        out_shape=jax.ShapeDtypeStruct((B, C), jnp.float32),
        compiler_params=pltpu.CompilerParams(
            dimension_semantics=("parallel", "arbitrary"),
            vmem_limit_bytes=56 * 1024 * 1024,
        ),
    )(xc, mw, mb, rw, rb)


@functools.partial(jax.jit, static_argnames=())
def kernel(x, model_weights, model_bias, resnet_weight, resnet_bias):
    B, D = x.shape
    M, _, C = model_weights.shape

    rw = _transpose_cast(resnet_weight.reshape(D, M, C))
    mb = model_bias.reshape(M, 1, C)
    rb = resnet_bias.reshape(M, 1, C)

    return _fused_call(x, model_weights, mb, rw, rb, min(B, 1024))


# speedup vs baseline: 1.0464x; 1.0464x over previous
import functools

import jax
import jax.numpy as jnp
from jax.experimental import pallas as pl
from jax.experimental.pallas import tpu as pltpu


def _fused_body(x_ref, w_ref, b_ref, rw_ref, rb_ref, o_ref):
    m = pl.program_id(1)
    xb = x_ref[...]
    logits = jnp.dot(xb, w_ref[0], preferred_element_type=jnp.float32)
    fusew = jnp.dot(xb, rw_ref[0], preferred_element_type=jnp.float32)
    term = (logits + b_ref[0]) * (fusew + rb_ref[0])

    @pl.when(m == 0)
    def _init():
        o_ref[...] = term

    @pl.when(m != 0)
    def _acc():
        o_ref[...] += term


def _fused_call(xc, mw, mb, rw, rb, bB):
    B, D = xc.shape
    M, _, C = mw.shape
    grid = (B // bB, M)
    return pl.pallas_call(
        _fused_body,
        grid=grid,
        in_specs=[
            pl.BlockSpec((bB, D), lambda b, m: (b, 0)),
            pl.BlockSpec((1, D, C), lambda b, m: (m, 0, 0)),
            pl.BlockSpec((1, 1, C), lambda b, m: (m, 0, 0)),
            pl.BlockSpec((1, D, C), lambda b, m: (m, 0, 0)),
            pl.BlockSpec((1, 1, C), lambda b, m: (m, 0, 0)),
        ],
        out_specs=pl.BlockSpec((bB, C), lambda b, m: (b, 0)),
        out_shape=jax.ShapeDtypeStruct((B, C), jnp.float32),
        compiler_params=pltpu.CompilerParams(
            dimension_semantics=("parallel", "arbitrary"),
            vmem_limit_bytes=56 * 1024 * 1024,
        ),
    )(xc, mw, mb, rw, rb)


@functools.partial(jax.jit, static_argnames=())
def kernel(x, model_weights, model_bias, resnet_weight, resnet_bias):
    B, D = x.shape
    M, _, C = model_weights.shape

    rw = resnet_weight.astype(jnp.bfloat16).reshape(D, M, C).transpose(1, 0, 2)
    mb = model_bias.reshape(M, 1, C)
    rb = resnet_bias.reshape(M, 1, C)

    return _fused_call(x, model_weights, mb, rw, rb, min(B, 1024))
